# baseline (speedup 1.0000x reference)
import functools

import jax
import jax.numpy as jnp
from jax import lax
from jax.experimental import pallas as pl
from jax.experimental.pallas import tpu as pltpu
from jax.experimental.pallas import tpu_sc as plsc

B = 16384
D = 64
NREL = 5
NC = 2
NS = 16
NW = NC * NS
CH = B // NW
G = 128
L = 16
KD = D // L

_GDN = lax.GatherDimensionNumbers(
    offset_dims=(), collapsed_slice_dims=(0,), start_index_map=(0,))


def _lane_shuffle(x, perm):
  return lax.gather(x, perm.reshape(L, 1), _GDN, (1,),
                    mode=lax.GatherScatterMode.PROMISE_IN_BOUNDS)


def _lane_sum(x):
  lanes = jnp.arange(L, dtype=jnp.int32)
  for sh in (8, 4, 2, 1):
    x = x + _lane_shuffle(x, lanes ^ sh)
  return x


def _hyperplane(rh_v, rel):
  h = [rh_v[NREL + rel, pl.ds(k * L, L)] for k in range(KD)]
  psq = h[0] * h[0]
  for k in range(1, KD):
    psq = psq + h[k] * h[k]
  scale = 1.0 / jnp.maximum(_lane_sum(psq), 1e-24)
  return h, scale


def _project(buf, h, scale):
  @plsc.parallel_loop(0, CH, unroll=4)
  def _(i):
    e = [buf[i, pl.ds(k * L, L)] for k in range(KD)]
    p = e[0] * h[0]
    for k in range(1, KD):
      p = p + e[k] * h[k]
    s = _lane_sum(p) * scale
    for k in range(KD):
      buf[i, pl.ds(k * L, L)] = e[k] - s * h[k]


def _doc_body(idx2, doc_t, relhyp,
              o_wrote, o_cited, o_rw, o_rc, o_rco, o_rv, o_ra,
              idx_v, buf, bcast, rh_v, sem, osem):
  wid = lax.axis_index("s") * NC + lax.axis_index("c")
  base = wid * CH

  pltpu.sync_copy(relhyp, rh_v)
  pltpu.sync_copy(idx2.at[:, pl.ds(base, CH)], idx_v)

  for r, out in enumerate((o_wrote, o_cited)):
    if r > 0:
      pltpu.make_async_copy(buf, o_wrote.at[pl.ds(base, CH)], osem).wait()

    @plsc.parallel_loop(0, CH, step=L)
    def _(i):
      rows = idx_v[r, pl.ds(i, L)]
      for j in range(L):
        pltpu.async_copy(
            doc_t.at[pl.ds(rows[j], 1)], buf.at[pl.ds(i + j, 1)], sem)

    @plsc.parallel_loop(0, CH, unroll=8)
    def _(i):
      pltpu.make_async_copy(
          doc_t.at[pl.ds(0, 1)], buf.at[pl.ds(0, 1)], sem).wait()

    h, scale = _hyperplane(rh_v, r)
    _project(buf, h, scale)

    pltpu.async_copy(buf, out.at[pl.ds(base, CH)], osem)

  bro = (o_rw, o_rc, o_rco, o_rv, o_ra)
  for rel in range(NREL):
    rv = [rh_v[rel, pl.ds(k * L, L)] for k in range(KD)]

    @plsc.parallel_loop(0, G, unroll=4)
    def _(i):
      for k in range(KD):
        bcast[i, pl.ds(k * L, L)] = rv[k]

    for j in range(CH // G):
      pltpu.sync_copy(bcast, bro[rel].at[pl.ds(base + j * G, G)])

  pltpu.make_async_copy(buf, o_cited.at[pl.ds(base, CH)], osem).wait()


def _small_body(idx4, user_t, venue_t, aff_t, relhyp,
                o_user, o_co, o_ven, o_aff,
                idx_v, bufa, bufb, rh_v, sema, semb, osem):
  wid = lax.axis_index("s") * NC + lax.axis_index("c")
  base = wid * CH

  pltpu.sync_copy(relhyp, rh_v)
  pltpu.sync_copy(idx4.at[:, pl.ds(base, CH)], idx_v)

  streams = (
      (user_t, o_user, None),
      (user_t, o_co, 2),
      (venue_t, o_ven, 3),
      (aff_t, o_aff, 4),
  )
  bufs = ((bufa, sema), (bufb, semb))

  def fire(r, buf, sem):
    tab = streams[r][0]
    for j in range(CH // G):
      pltpu.async_copy(
          tab.at[idx_v.at[r, pl.ds(j * G, G)]], buf.at[pl.ds(j * G, G)], sem)

  def drain_gather(r, buf, sem):
    tab = streams[r][0]
    for j in range(CH // G):
      pltpu.make_async_copy(
          tab.at[idx_v.at[r, pl.ds(0, G)]], buf.at[pl.ds(0, G)], sem).wait()

  fire(0, bufa, sema)

  for r, (tab, out, rel) in enumerate(streams):
    buf, sem = bufs[r % 2]
    if r + 1 < len(streams):
      nbuf, nsem = bufs[(r + 1) % 2]
      if r >= 1:
        pltpu.make_async_copy(
            nbuf, streams[r - 1][1].at[pl.ds(base, CH)], osem).wait()
      fire(r + 1, nbuf, nsem)
    drain_gather(r, buf, sem)

    if rel is not None:
      h, scale = _hyperplane(rh_v, rel)
      _project(buf, h, scale)

    pltpu.async_copy(buf, out.at[pl.ds(base, CH)], osem)

  pltpu.make_async_copy(bufa, o_ven.at[pl.ds(base, CH)], osem).wait()
  pltpu.make_async_copy(bufb, o_aff.at[pl.ds(base, CH)], osem).wait()


@jax.jit
def _run(uid, wrote, cited, coauth, ven, aff,
         user_t, venue_t, aff_t, doc_t, rel_t, hyp_t):
  relhyp = jnp.concatenate([rel_t, hyp_t])
  out = jax.ShapeDtypeStruct((B, D), jnp.float32)
  mesh = plsc.VectorSubcoreMesh(
      core_axis_name="c", subcore_axis_name="s", num_cores=NC, num_subcores=NS)

  o_wrote, o_cited, o_rw, o_rc, o_rco, o_rv, o_ra = pl.kernel(
      _doc_body,
      out_type=tuple(out for _ in range(7)),
      mesh=mesh,
      scratch_types=[
          pltpu.VMEM((2, CH), jnp.int32),
          pltpu.VMEM((CH, D), jnp.float32),
          pltpu.VMEM((G, D), jnp.float32),
          pltpu.VMEM((2 * NREL, D), jnp.float32),
          pltpu.SemaphoreType.DMA,
          pltpu.SemaphoreType.DMA,
      ],
  )(jnp.stack([wrote, cited]), doc_t, relhyp)

  o_user, o_co, o_ven, o_aff = pl.kernel(
      _small_body,
      out_type=tuple(out for _ in range(4)),
      mesh=mesh,
      compiler_params=pltpu.CompilerParams(use_tc_tiling_on_sc=False),
      scratch_types=[
          pltpu.VMEM((4, CH), jnp.int32),
          pltpu.VMEM((CH, D), jnp.float32),
          pltpu.VMEM((CH, D), jnp.float32),
          pltpu.VMEM((2 * NREL, D), jnp.float32),
          pltpu.SemaphoreType.DMA,
          pltpu.SemaphoreType.DMA,
          pltpu.SemaphoreType.DMA,
      ],
  )(jnp.stack([uid, coauth, ven, aff]), user_t, venue_t, aff_t, relhyp)

  return (o_user, o_wrote, o_cited, o_co, o_ven, o_aff,
          o_rw, o_rc, o_rco, o_rv, o_ra)


def kernel(user_id, wrote, cited, coauthor, venue, affiliation,
           user_table, venue_table, affiliation_table, doc_embedding,
           relation_table, hyper_plane):
  return _run(user_id, wrote, cited, coauthor, venue, affiliation,
              user_table, venue_table, affiliation_table, doc_embedding,
              relation_table, hyper_plane)


# speedup vs baseline: 1.0406x; 1.0406x over previous
import functools

import jax
import jax.numpy as jnp
from jax import lax
from jax.experimental import pallas as pl
from jax.experimental.pallas import tpu as pltpu
from jax.experimental.pallas import tpu_sc as plsc

B = 16384
D = 64
NREL = 5
NC = 2
NS = 16
NW = NC * NS
CH = B // NW
G = 128
L = 16
KD = D // L

_GDN = lax.GatherDimensionNumbers(
    offset_dims=(), collapsed_slice_dims=(0,), start_index_map=(0,))


def _lane_shuffle(x, perm):
  return lax.gather(x, perm.reshape(L, 1), _GDN, (1,),
                    mode=lax.GatherScatterMode.PROMISE_IN_BOUNDS)


def _lane_sum(x):
  lanes = jnp.arange(L, dtype=jnp.int32)
  for sh in (8, 4, 2, 1):
    x = x + _lane_shuffle(x, lanes ^ sh)
  return x


def _sc_body(uid, wrote, cited, coauth, ven, aff,
             user_t, venue_t, aff_t, doc_t, rel_t, hyp_t,
             o_user, o_wrote, o_cited, o_co, o_ven, o_aff,
             o_rw, o_rc, o_rco, o_rv, o_ra,
             idx_v, buf, bcast, hyp_v, rel_v, sem, osem):
  wid = lax.axis_index("s") * NC + lax.axis_index("c")
  base = wid * CH

  pltpu.sync_copy(hyp_t, hyp_v)
  pltpu.sync_copy(rel_t, rel_v)

  idx_in = (uid, wrote, cited, coauth, ven, aff)
  for r in range(6):
    pltpu.sync_copy(idx_in[r].at[pl.ds(base, CH)], idx_v.at[r])

  streams = (
      (user_t, o_user, None),
      (doc_t, o_wrote, 0),
      (doc_t, o_cited, 1),
      (user_t, o_co, 2),
      (venue_t, o_ven, 3),
      (aff_t, o_aff, 4),
  )

  def fire(r):
    tab = streams[r][0]

    @plsc.parallel_loop(0, CH, step=L)
    def _(i):
      rows = idx_v[r, pl.ds(i, L)]
      for j in range(L):
        pltpu.async_copy(
            tab.at[pl.ds(rows[j], 1)], buf.at[pl.ds(i + j, 1)], sem)

  fire(0)

  bro = (o_rw, o_rc, o_rco, o_rv, o_ra)
  for rel in range(NREL):
    rv = [rel_v[rel, pl.ds(k * L, L)] for k in range(KD)]

    @plsc.parallel_loop(0, G, unroll=4)
    def _(i):
      for k in range(KD):
        bcast[i, pl.ds(k * L, L)] = rv[k]

    for j in range(CH // G):
      pltpu.sync_copy(bcast, bro[rel].at[pl.ds(base + j * G, G)])

  for r, (tab, out, rel) in enumerate(streams):
    @plsc.parallel_loop(0, CH, unroll=8)
    def _(i):
      pltpu.make_async_copy(
          tab.at[pl.ds(0, 1)], buf.at[pl.ds(0, 1)], sem).wait()

    if rel is not None:
      h = [hyp_v[rel, pl.ds(k * L, L)] for k in range(KD)]
      psq = h[0] * h[0]
      for k in range(1, KD):
        psq = psq + h[k] * h[k]
      nsq = _lane_sum(psq)
      scale = 1.0 / jnp.maximum(nsq, 1e-24)

      @plsc.parallel_loop(0, CH, unroll=4)
      def _(i):
        e = [buf[i, pl.ds(k * L, L)] for k in range(KD)]
        p = e[0] * h[0]
        for k in range(1, KD):
          p = p + e[k] * h[k]
        s = _lane_sum(p) * scale
        for k in range(KD):
          buf[i, pl.ds(k * L, L)] = e[k] - s * h[k]

    pltpu.sync_copy(buf, out.at[pl.ds(base, CH)])
    if r + 1 < len(streams):
      fire(r + 1)


@jax.jit
def _run(uid, wrote, cited, coauth, ven, aff,
         user_t, venue_t, aff_t, doc_t, rel_t, hyp_t):
  out = jax.ShapeDtypeStruct((B, D), jnp.float32)
  mesh = plsc.VectorSubcoreMesh(
      core_axis_name="c", subcore_axis_name="s", num_cores=NC, num_subcores=NS)
  return pl.kernel(
      _sc_body,
      out_type=tuple(out for _ in range(11)),
      mesh=mesh,
      scratch_types=[
          pltpu.VMEM((6, CH), jnp.int32),
          pltpu.VMEM((CH, D), jnp.float32),
          pltpu.VMEM((G, D), jnp.float32),
          pltpu.VMEM((NREL, D), jnp.float32),
          pltpu.VMEM((NREL, D), jnp.float32),
          pltpu.SemaphoreType.DMA,
          pltpu.SemaphoreType.DMA,
      ],
  )(uid, wrote, cited, coauth, ven, aff,
    user_t, venue_t, aff_t, doc_t, rel_t, hyp_t)


def kernel(user_id, wrote, cited, coauthor, venue, affiliation,
           user_table, venue_table, affiliation_table, doc_embedding,
           relation_table, hyper_plane):
  return _run(user_id, wrote, cited, coauthor, venue, affiliation,
              user_table, venue_table, affiliation_table, doc_embedding,
              relation_table, hyper_plane)
